# baseline (speedup 1.0000x reference)
import functools

import jax
import jax.numpy as jnp
from jax import lax
from jax.experimental import pallas as pl
from jax.experimental.pallas import tpu as pltpu
from jax.experimental.pallas import tpu_sc as plsc

BATCH = 4096
SEQ = 200
DIM = 64
PDIM = 128
VOCAB2 = 2 * 1000000

NC = 2
NS = 16
NW = NC * NS
ROWS_PER_W = BATCH // NW
IDX_PER_W = ROWS_PER_W * SEQ
SUB = 128
SUBS_PER_CHUNK = 4
CHUNK = SUB * SUBS_PER_CHUNK
N_CHUNKS = IDX_PER_W // CHUNK
IDX_ROWS_PER_W = IDX_PER_W // SUB


def _sc_embed(ids2, dest, idn, zeros, table2):
    mesh = plsc.VectorSubcoreMesh(core_axis_name="c", subcore_axis_name="s")

    @functools.partial(
        pl.kernel,
        mesh=mesh,
        out_type=jax.ShapeDtypeStruct((BATCH, DIM), jnp.float32),
        compiler_params=pltpu.CompilerParams(use_tc_tiling_on_sc=False,
                                             needs_layout_passes=False),
        scratch_types=[
            pltpu.VMEM((2 * SUBS_PER_CHUNK, SUB), jnp.int32),
            pltpu.VMEM((2 * SUBS_PER_CHUNK, SUB), jnp.int32),
            pltpu.VMEM((1, SUB), jnp.int32),
            pltpu.VMEM((CHUNK, DIM), jnp.float32),
            pltpu.VMEM((CHUNK, DIM), jnp.float32),
            pltpu.VMEM((SUB, DIM), jnp.float32),
            pltpu.VMEM_SHARED((NS * ROWS_PER_W, DIM), jnp.float32),
            pltpu.SemaphoreType.DMA,
            pltpu.SemaphoreType.DMA,
            pltpu.SemaphoreType.DMA,
            pltpu.SemaphoreType.DMA,
        ],
    )
    def k(ids_hbm, dest_hbm, idn_hbm, zeros_hbm, table_hbm, out_hbm,
          idxb, destb, idnb, rows0, rows1, zbuf, acc,
          gsem0, gsem1, ssem, isem):
        cid = lax.axis_index("c")
        sid = lax.axis_index("s")
        wid = cid * NS + sid
        idx_base = wid * IDX_ROWS_PER_W
        accv = acc.at[pl.ds(sid * ROWS_PER_W, ROWS_PER_W)]

        pltpu.sync_copy(idn_hbm, idnb)
        pltpu.sync_copy(zeros_hbm, zbuf)
        pltpu.sync_copy(ids_hbm.at[pl.ds(idx_base, SUBS_PER_CHUNK)],
                        idxb.at[pl.ds(0, SUBS_PER_CHUNK)])
        pltpu.sync_copy(dest_hbm.at[pl.ds(0, SUBS_PER_CHUNK)],
                        destb.at[pl.ds(0, SUBS_PER_CHUNK)])

        pltpu.async_copy(zbuf, accv.at[idnb.at[0]], ssem).wait()

        def start_loads(c, p):
            pltpu.async_copy(
                ids_hbm.at[pl.ds(idx_base + c * SUBS_PER_CHUNK,
                                 SUBS_PER_CHUNK)],
                idxb.at[pl.ds(p * SUBS_PER_CHUNK, SUBS_PER_CHUNK)],
                isem)
            pltpu.async_copy(
                dest_hbm.at[pl.ds(c * SUBS_PER_CHUNK, SUBS_PER_CHUNK)],
                destb.at[pl.ds(p * SUBS_PER_CHUNK, SUBS_PER_CHUNK)],
                isem)

        def wait_loads(p):
            for ref, hbm in ((idxb, ids_hbm), (destb, dest_hbm)):
                pltpu.make_async_copy(
                    hbm.at[pl.ds(0, SUBS_PER_CHUNK)],
                    ref.at[pl.ds(p * SUBS_PER_CHUNK, SUBS_PER_CHUNK)],
                    isem).wait()

        def start_gather(p, buf, sem):
            for s in range(SUBS_PER_CHUNK):
                pltpu.async_copy(
                    table_hbm.at[idxb.at[p * SUBS_PER_CHUNK + s]],
                    buf.at[pl.ds(s * SUB, SUB)],
                    sem,
                )

        def wait_gather(buf, sem):
            for s in range(SUBS_PER_CHUNK):
                pltpu.make_async_copy(
                    table_hbm.at[idxb.at[s]],
                    buf.at[pl.ds(s * SUB, SUB)],
                    sem,
                ).wait()

        def scatter_chunk(p, buf):
            handles = []
            for s in range(SUBS_PER_CHUNK):
                handles.append(pltpu.async_copy(
                    buf.at[pl.ds(s * SUB, SUB)],
                    accv.at[destb.at[p * SUBS_PER_CHUNK + s]],
                    ssem,
                    add=True,
                ))
            for h in handles:
                h.wait()

        def process(c, p, buf, sem, nxt_buf, nxt_sem):
            @pl.when(c + 1 < N_CHUNKS)
            def _():
                wait_loads(1 - p)
                start_gather(1 - p, nxt_buf, nxt_sem)
            wait_gather(buf, sem)
            scatter_chunk(p, buf)

            @pl.when(c + 2 < N_CHUNKS)
            def _():
                start_loads(c + 2, p)

        start_loads(1, 1)
        start_gather(0, rows0, gsem0)

        def body(i, carry):
            c = i * 2
            process(c, 0, rows0, gsem0, rows1, gsem1)
            process(c + 1, 1, rows1, gsem1, rows0, gsem0)
            return carry

        lax.fori_loop(0, N_CHUNKS // 2, body, 0)

        pltpu.async_copy(zbuf, accv.at[idnb.at[0]], ssem, add=True).wait()
        pltpu.async_copy(accv.at[idnb.at[0]], rows0.at[pl.ds(0, SUB)],
                         gsem0).wait()

        def norm_row(r, carry):
            ss = jnp.zeros((16,), jnp.float32)
            for c4 in range(DIM // 16):
                x = rows0[r, pl.ds(c4 * 16, 16)]
                ss = ss + x * x
            s = jnp.sum(ss)
            s = jnp.maximum(s, jnp.float32((SEQ * 1e-12) ** 2))
            i = jax.lax.bitcast_convert_type(s, jnp.int32)
            i = jnp.int32(0x5F3759DF) - lax.shift_right_logical(i, 1)
            y = jax.lax.bitcast_convert_type(i, jnp.float32)
            for _ in range(3):
                y = y * (jnp.float32(1.5) - jnp.float32(0.5) * s * y * y)
            for c4 in range(DIM // 16):
                sl = pl.ds(c4 * 16, 16)
                rows0[r, sl] = rows0[r, sl] * y
            return carry

        lax.fori_loop(0, ROWS_PER_W, norm_row, 0)

        pltpu.sync_copy(rows0.at[pl.ds(0, ROWS_PER_W)],
                        out_hbm.at[pl.ds(wid * ROWS_PER_W, ROWS_PER_W)])

    return k(ids2, dest, idn, zeros, table2)


def kernel(input_ids, table):
    table2 = jnp.pad(table, ((0, 0), (0, PDIM - DIM))).reshape(VOCAB2, DIM)
    ids2 = (input_ids.astype(jnp.int32) * 2).reshape(-1, SUB)
    dest = (jnp.arange(IDX_PER_W, dtype=jnp.int32) // SEQ).reshape(
        IDX_ROWS_PER_W, SUB)
    idn = jnp.arange(SUB, dtype=jnp.int32)[None, :]
    zeros = jnp.zeros((SUB, DIM), jnp.float32)
    return _sc_embed(ids2, dest, idn, zeros, table2)


# speedup vs baseline: 1.0170x; 1.0170x over previous
import functools

import jax
import jax.numpy as jnp
from jax import lax
from jax.experimental import pallas as pl
from jax.experimental.pallas import tpu as pltpu
from jax.experimental.pallas import tpu_sc as plsc

BATCH = 4096
SEQ = 200
DIM = 64
PDIM = 128
VOCAB2 = 2 * 1000000

NC = 2
NS = 16
NW = NC * NS
ROWS_PER_W = BATCH // NW
IDX_PER_W = ROWS_PER_W * SEQ
SUB = 128
SUBS_PER_CHUNK = 2
CHUNK = SUB * SUBS_PER_CHUNK
N_CHUNKS = IDX_PER_W // CHUNK
IDX_ROWS_PER_W = IDX_PER_W // SUB


def _sc_pool(ids2, dest, zeros, table2):
    mesh = plsc.VectorSubcoreMesh(core_axis_name="c", subcore_axis_name="s")

    @functools.partial(
        pl.kernel,
        mesh=mesh,
        out_type=jax.ShapeDtypeStruct((BATCH, DIM), jnp.float32),
        compiler_params=pltpu.CompilerParams(use_tc_tiling_on_sc=False,
                                             needs_layout_passes=False),
        scratch_types=[
            pltpu.VMEM((IDX_ROWS_PER_W, SUB), jnp.int32),
            pltpu.VMEM((IDX_ROWS_PER_W + 1, SUB), jnp.int32),
            pltpu.VMEM((CHUNK, DIM), jnp.float32),
            pltpu.VMEM((CHUNK, DIM), jnp.float32),
            pltpu.VMEM((SUB, DIM), jnp.float32),
            pltpu.VMEM_SHARED((NS * ROWS_PER_W, DIM), jnp.float32),
            pltpu.SemaphoreType.DMA,
            pltpu.SemaphoreType.DMA,
            pltpu.SemaphoreType.DMA,
        ],
    )
    def k(ids_hbm, dest_hbm, zeros_hbm, table_hbm, out_hbm,
          idx_all, dest_all, rows0, rows1, zbuf, acc, gsem0, gsem1, ssem):
        cid = lax.axis_index("c")
        sid = lax.axis_index("s")
        wid = cid * NS + sid
        idx_base = wid * IDX_ROWS_PER_W

        pltpu.sync_copy(ids_hbm.at[pl.ds(idx_base, IDX_ROWS_PER_W)], idx_all)
        pltpu.sync_copy(dest_hbm, dest_all)
        pltpu.sync_copy(zeros_hbm, zbuf)
        off = (sid * ROWS_PER_W).astype(jnp.int32)

        def add_off(r, carry):
            for c4 in range(SUB // 16):
                sl = pl.ds(c4 * 16, 16)
                dest_all[r, sl] = dest_all[r, sl] + off
            return carry

        lax.fori_loop(0, IDX_ROWS_PER_W + 1, add_off, 0)

        pltpu.async_copy(zbuf, acc.at[dest_all.at[IDX_ROWS_PER_W]],
                         ssem).wait()

        def start_gather(c, buf, sem):
            for s in range(SUBS_PER_CHUNK):
                pltpu.async_copy(
                    table_hbm.at[idx_all.at[c * SUBS_PER_CHUNK + s]],
                    buf.at[pl.ds(s * SUB, SUB)],
                    sem,
                )

        def wait_gather(buf, sem):
            for s in range(SUBS_PER_CHUNK):
                pltpu.make_async_copy(
                    table_hbm.at[idx_all.at[s]],
                    buf.at[pl.ds(s * SUB, SUB)],
                    sem,
                ).wait()

        def scatter_chunk(c, buf):
            handles = []
            for s in range(SUBS_PER_CHUNK):
                handles.append(pltpu.async_copy(
                    buf.at[pl.ds(s * SUB, SUB)],
                    acc.at[dest_all.at[c * SUBS_PER_CHUNK + s]],
                    ssem,
                    add=True,
                ))
            for h in handles:
                h.wait()

        def process(c, buf, sem, nxt_buf, nxt_sem):
            @pl.when(c + 1 < N_CHUNKS)
            def _():
                start_gather(c + 1, nxt_buf, nxt_sem)
            wait_gather(buf, sem)
            scatter_chunk(c, buf)

        start_gather(0, rows0, gsem0)

        def body(i, carry):
            c = i * 2
            process(c, rows0, gsem0, rows1, gsem1)
            process(c + 1, rows1, gsem1, rows0, gsem0)
            return carry

        lax.fori_loop(0, N_CHUNKS // 2, body, 0)

        pltpu.async_copy(zbuf, acc.at[dest_all.at[IDX_ROWS_PER_W]],
                         ssem, add=True).wait()
        pltpu.async_copy(
            acc.at[dest_all.at[IDX_ROWS_PER_W]],
            rows0.at[pl.ds(0, SUB)],
            gsem0,
        ).wait()

        def norm_row(r, carry):
            ss = jnp.zeros((16,), jnp.float32)
            for c4 in range(DIM // 16):
                x = rows0[r, pl.ds(c4 * 16, 16)]
                ss = ss + x * x
            s = jnp.sum(ss)
            s = jnp.maximum(s, jnp.float32((SEQ * 1e-12) ** 2))
            i = jax.lax.bitcast_convert_type(s, jnp.int32)
            i = jnp.int32(0x5F3759DF) - lax.shift_right_logical(i, 1)
            y = jax.lax.bitcast_convert_type(i, jnp.float32)
            for _ in range(3):
                y = y * (jnp.float32(1.5) - jnp.float32(0.5) * s * y * y)
            for c4 in range(DIM // 16):
                sl = pl.ds(c4 * 16, 16)
                rows0[r, sl] = rows0[r, sl] * y
            return carry

        lax.fori_loop(0, ROWS_PER_W, norm_row, 0)

        pltpu.sync_copy(rows0.at[pl.ds(0, ROWS_PER_W)],
                        out_hbm.at[pl.ds(wid * ROWS_PER_W, ROWS_PER_W)])

    return k(ids2, dest, zeros, table2)


def kernel(input_ids, table):
    table2 = jnp.pad(table, ((0, 0), (0, PDIM - DIM))).reshape(VOCAB2, DIM)
    ids2 = (input_ids.astype(jnp.int32) * 2).reshape(-1, SUB)
    base = (jnp.arange(IDX_PER_W, dtype=jnp.int32) // SEQ).reshape(
        IDX_ROWS_PER_W, SUB)
    ident = jnp.arange(SUB, dtype=jnp.int32)[None, :]
    dest = jnp.concatenate([base, ident], axis=0)
    zeros = jnp.zeros((SUB, DIM), jnp.float32)
    return _sc_pool(ids2, dest, zeros, table2)
